# baseline (speedup 1.0000x reference)
import functools

import jax
import jax.numpy as jnp
from jax import lax
from jax.experimental import pallas as pl
from jax.experimental.pallas import tpu as pltpu
from jax.experimental.pallas import tpu_sc as plsc

E = 16
K = 2
D = 2048
FF = 1408
T = 2048

BROW = 256
NBLK = (T * K) // BROW + E
NROWS = NBLK * BROW
FFB = 128
NF = FF // FFB
TGATE = 256

NC = 2
NS = 16
NW = NC * NS
GCH = 16


def _gate_body(x_ref, wg_ref, tw_ref, ti_ref, xpk_ref):
    xb = x_ref[...]
    xbf = xb.astype(jnp.bfloat16)
    lo = lax.bitcast_convert_type(xbf[:, :D // 2], jnp.uint16).astype(
        jnp.uint32)
    hi = lax.bitcast_convert_type(xbf[:, D // 2:], jnp.uint16).astype(
        jnp.uint32)
    xpk_ref[...] = lax.bitcast_convert_type(lo | (hi << 16), jnp.int32)
    logits = lax.dot_general(xb, wg_ref[...], (((1,), (1,)), ((), ())),
                             preferred_element_type=jnp.float32)
    col = lax.broadcasted_iota(jnp.int32, logits.shape, 1)
    i0 = jnp.argmax(logits, axis=-1).astype(jnp.int32)
    m0 = jnp.max(logits, axis=-1)
    masked = jnp.where(col == i0[:, None], -jnp.inf, logits)
    i1 = jnp.argmax(masked, axis=-1).astype(jnp.int32)
    m1 = jnp.max(masked, axis=-1)
    w0 = 1.0 / (1.0 + jnp.exp(m1 - m0))
    w1 = 1.0 - w0
    ocol = lax.broadcasted_iota(jnp.int32, (TGATE, 128), 1)
    tw_ref[...] = jnp.where(ocol == 0, w0[:, None],
                            jnp.where(ocol == 1, w1[:, None], 0.0))
    ti_ref[...] = jnp.where(ocol == 0, i0[:, None],
                            jnp.where(ocol == 1, i1[:, None], 0))


def _gating(xf, Wg):
    return pl.pallas_call(
        _gate_body,
        grid=(T // TGATE,),
        in_specs=[
            pl.BlockSpec((TGATE, D), lambda i: (i, 0)),
            pl.BlockSpec((E, D), lambda i: (0, 0)),
        ],
        out_specs=[
            pl.BlockSpec((TGATE, 128), lambda i: (i, 0)),
            pl.BlockSpec((TGATE, 128), lambda i: (i, 0)),
            pl.BlockSpec((TGATE, D // 2), lambda i: (i, 0)),
        ],
        out_shape=[
            jax.ShapeDtypeStruct((T, 128), jnp.float32),
            jax.ShapeDtypeStruct((T, 128), jnp.int32),
            jax.ShapeDtypeStruct((T, D // 2), jnp.int32),
        ],
    )(xf, Wg)


def _route_meta(i0, i1, w0, w1):
    ef = jnp.concatenate([i0, i1])
    oh = (ef[:, None] == jnp.arange(E, dtype=jnp.int32)[None, :])
    csum = jnp.cumsum(oh.astype(jnp.int32), axis=0)
    rank = jnp.take_along_axis(csum, ef[:, None], axis=1)[:, 0] - 1
    counts = csum[-1]
    nb = (counts + BROW - 1) // BROW
    nb_cum = jnp.cumsum(nb)
    blk_first = nb_cum - nb
    pad_pos = blk_first[ef] * BROW + rank
    ar = jnp.arange(T, dtype=jnp.int32)
    tokid = jnp.concatenate([ar, ar])
    src_tok = jnp.zeros(NROWS, jnp.int32).at[pad_pos].set(tokid)
    w_pad = jnp.zeros(NROWS, jnp.float32).at[pad_pos].set(
        jnp.concatenate([w0, w1]))
    wrep = jnp.broadcast_to(w_pad.reshape(NBLK, BROW, 1), (NBLK, BROW, 8))
    j = jnp.arange(NBLK, dtype=jnp.int32)
    e_j = jnp.sum((j[:, None] >= nb_cum[None, :]).astype(jnp.int32), axis=1)
    validf = (j < nb_cum[-1]).astype(jnp.int32)
    e_j = jnp.minimum(e_j, E - 1) * validf
    meta = jnp.stack([e_j, validf]).astype(jnp.int32)
    return src_tok, wrep, meta, pad_pos[:T], pad_pos[T:]


D2 = D // 2


def _sc_gather(x32, src_tok):
    rows_per_w = NROWS // NW
    gch = 32
    niter = rows_per_w // gch

    @functools.partial(
        pl.kernel,
        out_type=jax.ShapeDtypeStruct((NROWS, D2), jnp.int32),
        mesh=plsc.VectorSubcoreMesh(core_axis_name="c", subcore_axis_name="s",
                                    num_cores=NC, num_subcores=NS),
        scratch_types=[
            [pltpu.VMEM((gch,), jnp.int32)] * 3,
            [pltpu.VMEM((gch, D2), jnp.int32)] * 3,
            [pltpu.SemaphoreType.DMA] * 3,
            [pltpu.SemaphoreType.DMA] * 3,
        ],
    )
    def gather_k(x_hbm, idx_hbm, xs_hbm, idx_v, buf_v, sg, st):
        wid = lax.axis_index("s") * NC + lax.axis_index("c")
        base = wid * rows_per_w

        def start_gather(c):
            m = c % 3
            pltpu.sync_copy(idx_hbm.at[pl.ds(base + c * gch, gch)], idx_v[m])
            pltpu.async_copy(x_hbm.at[idx_v[m]], buf_v[m], sg[m])

        def wait_gather(c):
            m = c % 3
            pltpu.make_async_copy(x_hbm.at[idx_v[m]], buf_v[m], sg[m]).wait()

        def start_store(c):
            m = c % 3
            pltpu.async_copy(buf_v[m], xs_hbm.at[pl.ds(base + c * gch, gch)],
                             st[m])

        def wait_store(c):
            m = c % 3
            pltpu.make_async_copy(
                buf_v[m], xs_hbm.at[pl.ds(base + c * gch, gch)], st[m]).wait()

        start_gather(0)
        start_gather(1)
        for c in range(niter):
            wait_gather(c)
            start_store(c)
            if c + 2 < niter:
                if c >= 1:
                    wait_store(c - 1)
                start_gather(c + 2)
        wait_store(niter - 2)
        wait_store(niter - 1)

    return gather_k(x32, src_tok)


def _wconv_body(w1_ref, w3_ref, w2_ref, o1_ref, o3_ref, o2_ref):
    o1_ref[...] = w1_ref[...].astype(jnp.bfloat16)
    o3_ref[...] = w3_ref[...].astype(jnp.bfloat16)
    o2_ref[...] = w2_ref[...].astype(jnp.bfloat16)


def _wconv(W1, W3, W2):
    FCH = FF // 8
    return pl.pallas_call(
        _wconv_body,
        grid=(E, 8),
        in_specs=[
            pl.BlockSpec((1, FCH, D), lambda e, f: (e, f, 0)),
            pl.BlockSpec((1, FCH, D), lambda e, f: (e, f, 0)),
            pl.BlockSpec((1, D // 8, FF), lambda e, f: (e, f, 0)),
        ],
        out_specs=[
            pl.BlockSpec((1, FCH, D), lambda e, f: (e, f, 0)),
            pl.BlockSpec((1, FCH, D), lambda e, f: (e, f, 0)),
            pl.BlockSpec((1, D // 8, FF), lambda e, f: (e, f, 0)),
        ],
        out_shape=[
            jax.ShapeDtypeStruct((E, FF, D), jnp.bfloat16),
            jax.ShapeDtypeStruct((E, FF, D), jnp.bfloat16),
            jax.ShapeDtypeStruct((E, D, FF), jnp.bfloat16),
        ],
    )(W1, W3, W2)


def _mm_body(meta_ref, xs_ref, w1_ref, w3_ref, w2_ref, wrep_ref, ys_ref):
    b = pl.program_id(0)

    @pl.when(meta_ref[1, b] > 0)
    def _():
        v = lax.bitcast_convert_type(xs_ref[...], jnp.uint32)
        lo = lax.bitcast_convert_type(
            (v & 0xFFFF).astype(jnp.uint16), jnp.bfloat16)
        hi = lax.bitcast_convert_type(
            (v >> 16).astype(jnp.uint16), jnp.bfloat16)
        xb = jnp.concatenate([lo, hi], axis=1)
        h1 = lax.dot_general(xb, w1_ref[0], (((1,), (1,)), ((), ())),
                             preferred_element_type=jnp.float32)
        h3 = lax.dot_general(xb, w3_ref[0], (((1,), (1,)), ((), ())),
                             preferred_element_type=jnp.float32)
        hc = ((h1 * lax.logistic(h1)) * h3).astype(jnp.bfloat16)
        yc = lax.dot_general(hc, w2_ref[0], (((1,), (1,)), ((), ())),
                             preferred_element_type=jnp.float32)
        ys_ref[...] = yc * wrep_ref[0, :, 0:1]


def _grouped_mlp(meta, xs, W1, W3, W2, wrep):
    grid_spec = pltpu.PrefetchScalarGridSpec(
        num_scalar_prefetch=1,
        grid=(NBLK,),
        in_specs=[
            pl.BlockSpec((BROW, D2), lambda b, m: (b, 0)),
            pl.BlockSpec((1, FF, D), lambda b, m: (m[0, b], 0, 0)),
            pl.BlockSpec((1, FF, D), lambda b, m: (m[0, b], 0, 0)),
            pl.BlockSpec((1, D, FF), lambda b, m: (m[0, b], 0, 0)),
            pl.BlockSpec((1, BROW, 8), lambda b, m: (b, 0, 0)),
        ],
        out_specs=pl.BlockSpec((BROW, D), lambda b, m: (b, 0)),
    )
    return pl.pallas_call(
        _mm_body,
        grid_spec=grid_spec,
        out_shape=jax.ShapeDtypeStruct((NROWS, D), jnp.float32),
        compiler_params=pltpu.CompilerParams(
            dimension_semantics=("arbitrary",)),
    )(meta, xs, W1, W3, W2, wrep)


CCH = 8


def _sc_combine(ys, pcat):
    toks_per_w = T // NW
    niter = toks_per_w // CCH

    @functools.partial(
        pl.kernel,
        out_type=jax.ShapeDtypeStruct((T, D), jnp.float32),
        mesh=plsc.VectorSubcoreMesh(core_axis_name="c", subcore_axis_name="s",
                                    num_cores=NC, num_subcores=NS),
        scratch_types=[
            pltpu.VMEM((2 * CCH,), jnp.int32),
            pltpu.VMEM((2 * CCH,), jnp.int32),
            pltpu.VMEM((2 * CCH, D), jnp.float32),
            pltpu.VMEM((2 * CCH, D), jnp.float32),
            pltpu.SemaphoreType.DMA,
            pltpu.SemaphoreType.DMA,
        ],
    )
    def combine_k(ys_hbm, pcat_hbm, out_hbm, i0_v, i1_v, r0_v, r1_v, s0, s1):
        wid = lax.axis_index("s") * NC + lax.axis_index("c")
        base = wid * toks_per_w

        def start(c, idx_v, buf_v, sem):
            pltpu.sync_copy(pcat_hbm.at[wid * niter + c], idx_v)
            pltpu.async_copy(ys_hbm.at[idx_v], buf_v, sem)

        def drain(c, idx_v, buf_v, sem):
            pltpu.make_async_copy(ys_hbm.at[idx_v], buf_v, sem).wait()

            def row(r, c2):
                for jj in range(D // 16):
                    sl = pl.ds(jj * 16, 16)
                    buf_v[r, sl] = buf_v[r, sl] + buf_v[r + CCH, sl]
                return c2

            lax.fori_loop(0, CCH, row, 0)
            pltpu.sync_copy(buf_v.at[pl.ds(0, CCH)],
                            out_hbm.at[pl.ds(base + c * CCH, CCH)])

        start(0, i0_v, r0_v, s0)

        def body(u, carry):
            c0 = 2 * u
            c1 = c0 + 1
            start(c1, i1_v, r1_v, s1)
            drain(c0, i0_v, r0_v, s0)

            @pl.when(c0 + 2 < niter)
            def _():
                start(c0 + 2, i0_v, r0_v, s0)

            drain(c1, i1_v, r1_v, s1)
            return carry

        lax.fori_loop(0, niter // 2, body, 0)

    return combine_k(ys, pcat)


def kernel(x, Wg, W1, W2, W3):
    xf = x.reshape(T, D)
    tw_pad, ti_pad, xpk = _gating(xf, Wg)
    i0 = ti_pad[:, 0]
    i1 = ti_pad[:, 1]
    w0 = tw_pad[:, 0]
    w1c = tw_pad[:, 1]
    src_tok, wrep, meta, p0, p1 = _route_meta(i0, i1, w0, w1c)
    pcat = jnp.concatenate(
        [p0.reshape(T // CCH, CCH), p1.reshape(T // CCH, CCH)], axis=1)
    w1b, w3b, w2b = _wconv(W1, W3, W2)
    xs32 = _sc_gather(xpk, src_tok)
    ys = _grouped_mlp(meta, xs32, w1b, w3b, w2b, wrep)
    out = _sc_combine(ys, pcat)
    return out.reshape(x.shape)


# speedup vs baseline: 1.0903x; 1.0903x over previous
import functools

import jax
import jax.numpy as jnp
from jax import lax
from jax.experimental import pallas as pl
from jax.experimental.pallas import tpu as pltpu
from jax.experimental.pallas import tpu_sc as plsc

E = 16
K = 2
D = 2048
FF = 1408
T = 2048

BROW = 256
NBLK = (T * K) // BROW + E
NROWS = NBLK * BROW
FFB = 128
NF = FF // FFB
TGATE = 256

NC = 2
NS = 16
NW = NC * NS
GCH = 16


def _gate_body(x_ref, wg_ref, tw_ref, ti_ref, xpk_ref):
    xb = x_ref[...]
    xbf = xb.astype(jnp.bfloat16)
    lo = lax.bitcast_convert_type(xbf[:, :D // 2], jnp.uint16).astype(
        jnp.uint32)
    hi = lax.bitcast_convert_type(xbf[:, D // 2:], jnp.uint16).astype(
        jnp.uint32)
    xpk_ref[...] = lax.bitcast_convert_type(lo | (hi << 16), jnp.int32)
    logits = lax.dot_general(xb, wg_ref[...], (((1,), (1,)), ((), ())),
                             preferred_element_type=jnp.float32)
    col = lax.broadcasted_iota(jnp.int32, logits.shape, 1)
    i0 = jnp.argmax(logits, axis=-1).astype(jnp.int32)
    m0 = jnp.max(logits, axis=-1)
    masked = jnp.where(col == i0[:, None], -jnp.inf, logits)
    i1 = jnp.argmax(masked, axis=-1).astype(jnp.int32)
    m1 = jnp.max(masked, axis=-1)
    w0 = 1.0 / (1.0 + jnp.exp(m1 - m0))
    w1 = 1.0 - w0
    ocol = lax.broadcasted_iota(jnp.int32, (TGATE, 128), 1)
    tw_ref[...] = jnp.where(ocol == 0, w0[:, None],
                            jnp.where(ocol == 1, w1[:, None], 0.0))
    ti_ref[...] = jnp.where(ocol == 0, i0[:, None],
                            jnp.where(ocol == 1, i1[:, None], 0))


def _gating(xf, Wg):
    return pl.pallas_call(
        _gate_body,
        grid=(T // TGATE,),
        in_specs=[
            pl.BlockSpec((TGATE, D), lambda i: (i, 0)),
            pl.BlockSpec((E, D), lambda i: (0, 0)),
        ],
        out_specs=[
            pl.BlockSpec((TGATE, 128), lambda i: (i, 0)),
            pl.BlockSpec((TGATE, 128), lambda i: (i, 0)),
            pl.BlockSpec((TGATE, D // 2), lambda i: (i, 0)),
        ],
        out_shape=[
            jax.ShapeDtypeStruct((T, 128), jnp.float32),
            jax.ShapeDtypeStruct((T, 128), jnp.int32),
            jax.ShapeDtypeStruct((T, D // 2), jnp.int32),
        ],
    )(xf, Wg)


def _route_meta(i0, i1, w0, w1):
    ef = jnp.concatenate([i0, i1])
    oh = (ef[:, None] == jnp.arange(E, dtype=jnp.int32)[None, :])
    csum = jnp.cumsum(oh.astype(jnp.int32), axis=0)
    rank = jnp.take_along_axis(csum, ef[:, None], axis=1)[:, 0] - 1
    counts = csum[-1]
    nb = (counts + BROW - 1) // BROW
    nb_cum = jnp.cumsum(nb)
    blk_first = nb_cum - nb
    pad_pos = blk_first[ef] * BROW + rank
    ar = jnp.arange(T, dtype=jnp.int32)
    tokid = jnp.concatenate([ar, ar])
    src_tok = jnp.zeros(NROWS, jnp.int32).at[pad_pos].set(tokid)
    w_pad = jnp.zeros(NROWS, jnp.float32).at[pad_pos].set(
        jnp.concatenate([w0, w1]))
    wrep = jnp.broadcast_to(w_pad.reshape(NBLK, BROW, 1), (NBLK, BROW, 8))
    j = jnp.arange(NBLK, dtype=jnp.int32)
    e_j = jnp.sum((j[:, None] >= nb_cum[None, :]).astype(jnp.int32), axis=1)
    validf = (j < nb_cum[-1]).astype(jnp.int32)
    e_j = jnp.minimum(e_j, E - 1) * validf
    meta = jnp.stack([e_j, validf]).astype(jnp.int32)
    return src_tok, wrep, meta, pad_pos[:T], pad_pos[T:]


D2 = D // 2


def _sc_gather(x32, src_tok):
    rows_per_w = NROWS // NW
    gch = 32
    niter = rows_per_w // gch

    @functools.partial(
        pl.kernel,
        out_type=jax.ShapeDtypeStruct((NROWS, D2), jnp.int32),
        mesh=plsc.VectorSubcoreMesh(core_axis_name="c", subcore_axis_name="s",
                                    num_cores=NC, num_subcores=NS),
        scratch_types=[
            [pltpu.VMEM((gch,), jnp.int32)] * 3,
            [pltpu.VMEM((gch, D2), jnp.int32)] * 3,
            [pltpu.SemaphoreType.DMA] * 3,
            [pltpu.SemaphoreType.DMA] * 3,
        ],
    )
    def gather_k(x_hbm, idx_hbm, xs_hbm, idx_v, buf_v, sg, st):
        wid = lax.axis_index("s") * NC + lax.axis_index("c")
        base = wid * rows_per_w

        def start_gather(c):
            m = c % 3
            pltpu.sync_copy(idx_hbm.at[pl.ds(base + c * gch, gch)], idx_v[m])
            pltpu.async_copy(x_hbm.at[idx_v[m]], buf_v[m], sg[m])

        def wait_gather(c):
            m = c % 3
            pltpu.make_async_copy(x_hbm.at[idx_v[m]], buf_v[m], sg[m]).wait()

        def start_store(c):
            m = c % 3
            pltpu.async_copy(buf_v[m], xs_hbm.at[pl.ds(base + c * gch, gch)],
                             st[m])

        def wait_store(c):
            m = c % 3
            pltpu.make_async_copy(
                buf_v[m], xs_hbm.at[pl.ds(base + c * gch, gch)], st[m]).wait()

        start_gather(0)
        start_gather(1)
        for c in range(niter):
            wait_gather(c)
            start_store(c)
            if c + 2 < niter:
                if c >= 1:
                    wait_store(c - 1)
                start_gather(c + 2)
        wait_store(niter - 2)
        wait_store(niter - 1)

    return gather_k(x32, src_tok)


def _wconv_body(w1_ref, w3_ref, o1_ref, o3_ref):
    o1_ref[...] = w1_ref[...].astype(jnp.bfloat16)
    o3_ref[...] = w3_ref[...].astype(jnp.bfloat16)


def _wconv(W1, W3):
    FCH = FF // 8
    return pl.pallas_call(
        _wconv_body,
        grid=(E, 8),
        in_specs=[
            pl.BlockSpec((1, FCH, D), lambda e, f: (e, f, 0)),
            pl.BlockSpec((1, FCH, D), lambda e, f: (e, f, 0)),
        ],
        out_specs=[
            pl.BlockSpec((1, FCH, D), lambda e, f: (e, f, 0)),
            pl.BlockSpec((1, FCH, D), lambda e, f: (e, f, 0)),
        ],
        out_shape=[
            jax.ShapeDtypeStruct((E, FF, D), jnp.bfloat16),
            jax.ShapeDtypeStruct((E, FF, D), jnp.bfloat16),
        ],
    )(W1, W3)


def _mm_body(meta_ref, xs_ref, w1_ref, w3_ref, w2_ref, wrep_ref, ys_ref):
    b = pl.program_id(0)

    @pl.when(meta_ref[1, b] > 0)
    def _():
        v = lax.bitcast_convert_type(xs_ref[...], jnp.uint32)
        lo = lax.bitcast_convert_type(
            (v & 0xFFFF).astype(jnp.uint16), jnp.bfloat16)
        hi = lax.bitcast_convert_type(
            (v >> 16).astype(jnp.uint16), jnp.bfloat16)
        xb = jnp.concatenate([lo, hi], axis=1)
        h1 = lax.dot_general(xb, w1_ref[0], (((1,), (1,)), ((), ())),
                             preferred_element_type=jnp.float32)
        h3 = lax.dot_general(xb, w3_ref[0], (((1,), (1,)), ((), ())),
                             preferred_element_type=jnp.float32)
        hc = ((h1 * lax.logistic(h1)) * h3).astype(jnp.bfloat16)
        w2b = w2_ref[0].astype(jnp.bfloat16)
        yc = lax.dot_general(hc, w2b, (((1,), (1,)), ((), ())),
                             preferred_element_type=jnp.float32)
        ys_ref[...] = yc * wrep_ref[0, :, 0:1]


def _grouped_mlp(meta, xs, W1, W3, W2, wrep):
    grid_spec = pltpu.PrefetchScalarGridSpec(
        num_scalar_prefetch=1,
        grid=(NBLK,),
        in_specs=[
            pl.BlockSpec((BROW, D2), lambda b, m: (b, 0)),
            pl.BlockSpec((1, FF, D), lambda b, m: (m[0, b], 0, 0)),
            pl.BlockSpec((1, FF, D), lambda b, m: (m[0, b], 0, 0)),
            pl.BlockSpec((1, D, FF), lambda b, m: (m[0, b], 0, 0)),
            pl.BlockSpec((1, BROW, 8), lambda b, m: (b, 0, 0)),
        ],
        out_specs=pl.BlockSpec((BROW, D), lambda b, m: (b, 0)),
    )
    return pl.pallas_call(
        _mm_body,
        grid_spec=grid_spec,
        out_shape=jax.ShapeDtypeStruct((NROWS, D), jnp.float32),
        compiler_params=pltpu.CompilerParams(
            dimension_semantics=("arbitrary",)),
    )(meta, xs, W1, W3, W2, wrep)


CCH = 8


def _sc_combine(ys, pcat):
    toks_per_w = T // NW
    niter = toks_per_w // CCH

    @functools.partial(
        pl.kernel,
        out_type=jax.ShapeDtypeStruct((T, D), jnp.float32),
        mesh=plsc.VectorSubcoreMesh(core_axis_name="c", subcore_axis_name="s",
                                    num_cores=NC, num_subcores=NS),
        scratch_types=[
            pltpu.VMEM((2 * CCH,), jnp.int32),
            pltpu.VMEM((2 * CCH,), jnp.int32),
            pltpu.VMEM((2 * CCH, D), jnp.float32),
            pltpu.VMEM((2 * CCH, D), jnp.float32),
            pltpu.SemaphoreType.DMA,
            pltpu.SemaphoreType.DMA,
        ],
    )
    def combine_k(ys_hbm, pcat_hbm, out_hbm, i0_v, i1_v, r0_v, r1_v, s0, s1):
        wid = lax.axis_index("s") * NC + lax.axis_index("c")
        base = wid * toks_per_w

        def start(c, idx_v, buf_v, sem):
            pltpu.sync_copy(pcat_hbm.at[wid * niter + c], idx_v)
            pltpu.async_copy(ys_hbm.at[idx_v], buf_v, sem)

        def drain(c, idx_v, buf_v, sem):
            pltpu.make_async_copy(ys_hbm.at[idx_v], buf_v, sem).wait()

            def row(r, c2):
                for jj in range(D // 16):
                    sl = pl.ds(jj * 16, 16)
                    buf_v[r, sl] = buf_v[r, sl] + buf_v[r + CCH, sl]
                return c2

            lax.fori_loop(0, CCH, row, 0)
            pltpu.sync_copy(buf_v.at[pl.ds(0, CCH)],
                            out_hbm.at[pl.ds(base + c * CCH, CCH)])

        start(0, i0_v, r0_v, s0)

        def body(u, carry):
            c0 = 2 * u
            c1 = c0 + 1
            start(c1, i1_v, r1_v, s1)
            drain(c0, i0_v, r0_v, s0)

            @pl.when(c0 + 2 < niter)
            def _():
                start(c0 + 2, i0_v, r0_v, s0)

            drain(c1, i1_v, r1_v, s1)
            return carry

        lax.fori_loop(0, niter // 2, body, 0)

    return combine_k(ys, pcat)


def kernel(x, Wg, W1, W2, W3):
    xf = x.reshape(T, D)
    tw_pad, ti_pad, xpk = _gating(xf, Wg)
    i0 = ti_pad[:, 0]
    i1 = ti_pad[:, 1]
    w0 = tw_pad[:, 0]
    w1c = tw_pad[:, 1]
    src_tok, wrep, meta, p0, p1 = _route_meta(i0, i1, w0, w1c)
    pcat = jnp.concatenate(
        [p0.reshape(T // CCH, CCH), p1.reshape(T // CCH, CCH)], axis=1)
    w1b, w3b = _wconv(W1, W3)
    xs32 = _sc_gather(xpk, src_tok)
    ys = _grouped_mlp(meta, xs32, w1b, w3b, W2, wrep)
    out = _sc_combine(ys, pcat)
    return out.reshape(x.shape)
